# baseline (speedup 1.0000x reference)
import functools

import jax
import jax.numpy as jnp
from jax import lax
from jax.experimental import pallas as pl
from jax.experimental.pallas import tpu as pltpu
from jax.experimental.pallas import tpu_sc as plsc

NC = 2
NS = 16
NW = NC * NS
EDGE_CHUNK = 80


def _make_sc_aggregate(n_pad, n_edges, d, with_deg):
    rows_per_tile = n_pad // NS
    assert rows_per_tile % EDGE_CHUNK == 0
    edges_per_worker = n_edges // NW
    assert edges_per_worker % EDGE_CHUNK == 0
    n_chunks = edges_per_worker // EDGE_CHUNK
    assert n_chunks % 2 == 1 and n_chunks >= 3

    mesh = plsc.VectorSubcoreMesh(core_axis_name="c", subcore_axis_name="s")

    stage_dst = not with_deg

    out_type = [jax.ShapeDtypeStruct((NC, n_pad, d), jnp.float32)]
    scratch = [
        pltpu.VMEM((edges_per_worker,), jnp.int32),
        pltpu.VMEM((EDGE_CHUNK,), jnp.int32),
        pltpu.VMEM((EDGE_CHUNK,), jnp.int32),
        pltpu.VMEM((EDGE_CHUNK, d), jnp.float32),
        pltpu.VMEM((EDGE_CHUNK, d), jnp.float32),
        pltpu.VMEM_SHARED((n_pad, d), jnp.float32),
        pltpu.SemaphoreType.DMA,
        pltpu.SemaphoreType.DMA,
    ]
    if with_deg:
        out_type.append(jax.ShapeDtypeStruct((NW * n_pad,), jnp.float32))
        scratch.append(pltpu.VMEM((n_pad,), jnp.float32))
    if stage_dst:
        scratch.append(pltpu.VMEM((edges_per_worker,), jnp.int32))

    @functools.partial(
        pl.kernel, mesh=mesh, out_type=out_type, scratch_types=scratch,
        compiler_params=pltpu.CompilerParams(needs_layout_passes=False))
    def sc_aggregate(rows_hbm, src_hbm, dst_hbm, out_hbm, *rest):
        if with_deg:
            (deg_hbm, src_all, dst_a, dst_b, rows_a, rows_b,
             agg_sh, sem_a, sem_b, deg_v) = rest
            dst_all = None
        else:
            (src_all, dst_a, dst_b, rows_a, rows_b, agg_sh,
             sem_a, sem_b, dst_all) = rest
        c = lax.axis_index("c")
        s = lax.axis_index("s")
        w = c * NS + s
        base = w * edges_per_worker

        pltpu.sync_copy(src_hbm.at[pl.ds(base, edges_per_worker)], src_all)
        if stage_dst:
            pltpu.sync_copy(dst_hbm.at[pl.ds(base, edges_per_worker)],
                            dst_all)

        def _zrow(i, _):
            def _zcol(j, _):
                rows_a[i, pl.ds(j * 16, 16)] = jnp.zeros((16,), jnp.float32)
                return 0
            return lax.fori_loop(0, d // 16, _zcol, 0)
        lax.fori_loop(0, EDGE_CHUNK, _zrow, 0)

        def _zcopy(k, _):
            pltpu.sync_copy(
                rows_a,
                agg_sh.at[pl.ds(s * rows_per_tile + k * EDGE_CHUNK,
                                EDGE_CHUNK)])
            return 0
        lax.fori_loop(0, rows_per_tile // EDGE_CHUNK, _zcopy, 0)

        if with_deg:
            def _zdeg(i, _):
                deg_v[pl.ds(i * 16, 16)] = jnp.zeros((16,), jnp.float32)
                return 0
            lax.fori_loop(0, n_pad // 16, _zdeg, 0)
            ones16 = jnp.ones((16,), jnp.float32)

        plsc.subcore_barrier()

        H1 = 48
        H2 = EDGE_CHUNK - H1

        def _gather(j, buf, sem):
            pltpu.async_copy(
                rows_hbm.at[src_all.at[pl.ds(j * EDGE_CHUNK, H1)]],
                buf.at[pl.ds(0, H1)], sem)
            pltpu.async_copy(
                rows_hbm.at[src_all.at[pl.ds(j * EDGE_CHUNK + H1, H2)]],
                buf.at[pl.ds(H1, H2)], sem)

        def _wait(buf, sem):
            pltpu.make_async_copy(rows_hbm.at[pl.ds(0, H1)],
                                  buf.at[pl.ds(0, H1)], sem).wait()
            pltpu.make_async_copy(rows_hbm.at[pl.ds(0, H2)],
                                  buf.at[pl.ds(H1, H2)], sem).wait()

        def _scatter(dst_v, buf):
            pltpu.sync_copy(buf, agg_sh.at[dst_v], add=True)

        def _load_dst(j, dst_v):
            if stage_dst:
                for k in range(EDGE_CHUNK // 16):
                    dst_v[pl.ds(k * 16, 16)] = (
                        dst_all[pl.ds(j * EDGE_CHUNK + k * 16, 16)])
            else:
                pltpu.sync_copy(dst_hbm.at[pl.ds(base + j * EDGE_CHUNK,
                                                 EDGE_CHUNK)], dst_v)
            if with_deg:
                for k in range(EDGE_CHUNK // 16):
                    idx16 = dst_v[pl.ds(k * 16, 16)]
                    plsc.addupdate_scatter(deg_v, [idx16], ones16)

        _gather(0, rows_a, sem_a)
        _load_dst(0, dst_a)

        def _pair(gg, _):
            j0 = 2 * gg
            _gather(j0 + 1, rows_b, sem_b)
            _load_dst(j0 + 1, dst_b)
            _wait(rows_a, sem_a)
            _scatter(dst_a, rows_a)
            _gather(j0 + 2, rows_a, sem_a)
            _load_dst(j0 + 2, dst_a)
            _wait(rows_b, sem_b)
            _scatter(dst_b, rows_b)
            return 0
        lax.fori_loop(0, (n_chunks - 1) // 2, _pair, 0)

        _wait(rows_a, sem_a)
        _scatter(dst_a, rows_a)

        if with_deg:
            pltpu.sync_copy(deg_v, deg_hbm.at[pl.ds(w * n_pad, n_pad)])

        plsc.subcore_barrier()

        pltpu.sync_copy(agg_sh.at[pl.ds(s * rows_per_tile, rows_per_tile)],
                        out_hbm.at[c, pl.ds(s * rows_per_tile, rows_per_tile)])

    return sc_aggregate


def _deg_column(dp):
    dpt = jnp.transpose(dp)
    return jnp.maximum(jnp.sum(dpt, axis=1, keepdims=True), 1.0)


def _tc_layer1(x, p, degp, w_self, w_neigh, b, blk):
    n, din = x.shape

    def body(x_ref, p_ref, dp_ref, ws_ref, wn_ref, b_ref, h_ref):
        agg = p_ref[0] + p_ref[1]
        hn = agg / _deg_column(dp_ref[...])
        h = (jnp.dot(x_ref[...], ws_ref[...], preferred_element_type=jnp.float32)
             + jnp.dot(hn, wn_ref[...], preferred_element_type=jnp.float32)
             + b_ref[...])
        h_ref[...] = jnp.maximum(h, 0.0)

    return pl.pallas_call(
        body,
        grid=(n // blk,),
        in_specs=[
            pl.BlockSpec((blk, din), lambda i: (i, 0)),
            pl.BlockSpec((NC, blk, din), lambda i: (0, i, 0)),
            pl.BlockSpec((NW, blk), lambda i: (0, i)),
            pl.BlockSpec((din, din), lambda i: (0, 0)),
            pl.BlockSpec((din, din), lambda i: (0, 0)),
            pl.BlockSpec((1, din), lambda i: (0, 0)),
        ],
        out_specs=pl.BlockSpec((blk, din), lambda i: (i, 0)),
        out_shape=jax.ShapeDtypeStruct((n, din), jnp.float32),
    )(x, p, degp, w_self, w_neigh, b)


def _tc_layer2(h1, q, degp, w_self, w_neigh, b, blk):
    n, d = h1.shape

    def body(h_ref, q_ref, dp_ref, ws_ref, wn_ref, b_ref, o_ref):
        hn = (q_ref[0] + q_ref[1]) / _deg_column(dp_ref[...])
        o_ref[...] = (
            jnp.dot(h_ref[...], ws_ref[...], preferred_element_type=jnp.float32)
            + jnp.dot(hn, wn_ref[...], preferred_element_type=jnp.float32)
            + b_ref[...])

    return pl.pallas_call(
        body,
        grid=(n // blk,),
        in_specs=[
            pl.BlockSpec((blk, d), lambda i: (i, 0)),
            pl.BlockSpec((NC, blk, d), lambda i: (0, i, 0)),
            pl.BlockSpec((NW, blk), lambda i: (0, i)),
            pl.BlockSpec((d, d), lambda i: (0, 0)),
            pl.BlockSpec((d, d), lambda i: (0, 0)),
            pl.BlockSpec((1, d), lambda i: (0, 0)),
        ],
        out_specs=pl.BlockSpec((blk, d), lambda i: (i, 0)),
        out_shape=jax.ShapeDtypeStruct((n, d), jnp.float32),
    )(h1, q, degp, w_self, w_neigh, b)


def kernel(x, edge_index, W_self1, W_neigh1, b1, W_self2, W_neigh2, b2):
    n, din = x.shape
    e = edge_index.shape[1]
    src = edge_index[0].astype(jnp.int32)
    dst = edge_index[1].astype(jnp.int32)

    blk = 2048
    n_pad = ((n + NS * 128 - 1) // (NS * 128)) * (NS * 128)
    xp = jnp.zeros((n_pad, din), jnp.float32).at[:n].set(x)

    p, degf = _make_sc_aggregate(n_pad, e, din, True)(xp, src, dst)
    degp = degf.reshape(NW, n_pad)
    h1 = _tc_layer1(xp, p, degp, W_self1, W_neigh1, b1.reshape(1, -1), blk)
    (q,) = _make_sc_aggregate(n_pad, e, din, False)(h1, src, dst)
    out = _tc_layer2(h1, q, degp, W_self2, W_neigh2, b2.reshape(1, -1), blk)
    return out[:n]


# speedup vs baseline: 1.0020x; 1.0020x over previous
import functools

import jax
import jax.numpy as jnp
from jax import lax
from jax.experimental import pallas as pl
from jax.experimental.pallas import tpu as pltpu
from jax.experimental.pallas import tpu_sc as plsc

NC = 2
NS = 16
NW = NC * NS
EDGE_CHUNK = 80


def _make_sc_aggregate(n_pad, n_edges, d, with_deg):
    rows_per_tile = n_pad // NS
    assert rows_per_tile % EDGE_CHUNK == 0
    edges_per_worker = n_edges // NW
    assert edges_per_worker % EDGE_CHUNK == 0
    n_chunks = edges_per_worker // EDGE_CHUNK
    assert n_chunks % 2 == 1 and n_chunks >= 3

    mesh = plsc.VectorSubcoreMesh(core_axis_name="c", subcore_axis_name="s")

    stage_dst = not with_deg

    out_type = [jax.ShapeDtypeStruct((NC, n_pad, d), jnp.float32)]
    scratch = [
        pltpu.VMEM((edges_per_worker,), jnp.int32),
        pltpu.VMEM((EDGE_CHUNK,), jnp.int32),
        pltpu.VMEM((EDGE_CHUNK,), jnp.int32),
        pltpu.VMEM((EDGE_CHUNK, d), jnp.float32),
        pltpu.VMEM((EDGE_CHUNK, d), jnp.float32),
        pltpu.VMEM_SHARED((n_pad, d), jnp.float32),
        pltpu.SemaphoreType.DMA,
        pltpu.SemaphoreType.DMA,
    ]
    if with_deg:
        out_type.append(jax.ShapeDtypeStruct((NW * n_pad,), jnp.float32))
        scratch.append(pltpu.VMEM((n_pad,), jnp.float32))
    if stage_dst:
        scratch.append(pltpu.VMEM((edges_per_worker,), jnp.int32))

    @functools.partial(
        pl.kernel, mesh=mesh, out_type=out_type, scratch_types=scratch,
        compiler_params=pltpu.CompilerParams(needs_layout_passes=False))
    def sc_aggregate(rows_hbm, src_hbm, dst_hbm, out_hbm, *rest):
        if with_deg:
            (deg_hbm, src_all, dst_a, dst_b, rows_a, rows_b,
             agg_sh, sem_a, sem_b, deg_v) = rest
            dst_all = None
        else:
            (src_all, dst_a, dst_b, rows_a, rows_b, agg_sh,
             sem_a, sem_b, dst_all) = rest
        c = lax.axis_index("c")
        s = lax.axis_index("s")
        w = c * NS + s
        base = w * edges_per_worker

        pltpu.sync_copy(src_hbm.at[pl.ds(base, edges_per_worker)], src_all)
        if stage_dst:
            pltpu.sync_copy(dst_hbm.at[pl.ds(base, edges_per_worker)],
                            dst_all)

        def _zrow(i, _):
            def _zcol(j, _):
                rows_a[i, pl.ds(j * 16, 16)] = jnp.zeros((16,), jnp.float32)
                return 0
            return lax.fori_loop(0, d // 16, _zcol, 0)
        lax.fori_loop(0, EDGE_CHUNK, _zrow, 0)

        def _zcopy(k, _):
            pltpu.sync_copy(
                rows_a,
                agg_sh.at[pl.ds(s * rows_per_tile + k * EDGE_CHUNK,
                                EDGE_CHUNK)])
            return 0
        lax.fori_loop(0, rows_per_tile // EDGE_CHUNK, _zcopy, 0)

        if with_deg:
            def _zdeg(i, _):
                deg_v[pl.ds(i * 16, 16)] = jnp.zeros((16,), jnp.float32)
                return 0
            lax.fori_loop(0, n_pad // 16, _zdeg, 0)
            ones16 = jnp.ones((16,), jnp.float32)

        plsc.subcore_barrier()

        def _gather(j, buf, sem):
            pltpu.async_copy(
                rows_hbm.at[src_all.at[pl.ds(j * EDGE_CHUNK, EDGE_CHUNK)]],
                buf, sem)

        def _wait(buf, sem):
            pltpu.make_async_copy(rows_hbm.at[pl.ds(0, EDGE_CHUNK)], buf,
                                  sem).wait()

        def _scatter(dst_v, buf):
            pltpu.sync_copy(buf, agg_sh.at[dst_v], add=True)

        def _load_dst(j, dst_v):
            if stage_dst:
                for k in range(EDGE_CHUNK // 16):
                    dst_v[pl.ds(k * 16, 16)] = (
                        dst_all[pl.ds(j * EDGE_CHUNK + k * 16, 16)])
            else:
                pltpu.sync_copy(dst_hbm.at[pl.ds(base + j * EDGE_CHUNK,
                                                 EDGE_CHUNK)], dst_v)
            if with_deg:
                for k in range(EDGE_CHUNK // 16):
                    idx16 = dst_v[pl.ds(k * 16, 16)]
                    plsc.addupdate_scatter(deg_v, [idx16], ones16)

        _gather(0, rows_a, sem_a)
        _load_dst(0, dst_a)

        def _pair(gg, _):
            j0 = 2 * gg
            _gather(j0 + 1, rows_b, sem_b)
            _load_dst(j0 + 1, dst_b)
            _wait(rows_a, sem_a)
            _scatter(dst_a, rows_a)
            _gather(j0 + 2, rows_a, sem_a)
            _load_dst(j0 + 2, dst_a)
            _wait(rows_b, sem_b)
            _scatter(dst_b, rows_b)
            return 0
        lax.fori_loop(0, (n_chunks - 1) // 2, _pair, 0)

        _wait(rows_a, sem_a)
        _scatter(dst_a, rows_a)

        if with_deg:
            pltpu.sync_copy(deg_v, deg_hbm.at[pl.ds(w * n_pad, n_pad)])

        plsc.subcore_barrier()

        pltpu.sync_copy(agg_sh.at[pl.ds(s * rows_per_tile, rows_per_tile)],
                        out_hbm.at[c, pl.ds(s * rows_per_tile, rows_per_tile)])

    return sc_aggregate


def _deg_column(dp):
    dpt = jnp.transpose(dp)
    return jnp.maximum(jnp.sum(dpt, axis=1, keepdims=True), 1.0)


def _tc_layer1(x, p, degp, w_self, w_neigh, b, blk):
    n, din = x.shape

    def body(x_ref, p_ref, dp_ref, ws_ref, wn_ref, b_ref, h_ref):
        agg = p_ref[0] + p_ref[1]
        hn = agg / _deg_column(dp_ref[...])
        h = (jnp.dot(x_ref[...], ws_ref[...], preferred_element_type=jnp.float32)
             + jnp.dot(hn, wn_ref[...], preferred_element_type=jnp.float32)
             + b_ref[...])
        h_ref[...] = jnp.maximum(h, 0.0)

    return pl.pallas_call(
        body,
        grid=(n // blk,),
        in_specs=[
            pl.BlockSpec((blk, din), lambda i: (i, 0)),
            pl.BlockSpec((NC, blk, din), lambda i: (0, i, 0)),
            pl.BlockSpec((NW, blk), lambda i: (0, i)),
            pl.BlockSpec((din, din), lambda i: (0, 0)),
            pl.BlockSpec((din, din), lambda i: (0, 0)),
            pl.BlockSpec((1, din), lambda i: (0, 0)),
        ],
        out_specs=pl.BlockSpec((blk, din), lambda i: (i, 0)),
        out_shape=jax.ShapeDtypeStruct((n, din), jnp.float32),
    )(x, p, degp, w_self, w_neigh, b)


def _tc_layer2(h1, q, degp, w_self, w_neigh, b, blk):
    n, d = h1.shape

    def body(h_ref, q_ref, dp_ref, ws_ref, wn_ref, b_ref, o_ref):
        hn = (q_ref[0] + q_ref[1]) / _deg_column(dp_ref[...])
        o_ref[...] = (
            jnp.dot(h_ref[...], ws_ref[...], preferred_element_type=jnp.float32)
            + jnp.dot(hn, wn_ref[...], preferred_element_type=jnp.float32)
            + b_ref[...])

    return pl.pallas_call(
        body,
        grid=(n // blk,),
        in_specs=[
            pl.BlockSpec((blk, d), lambda i: (i, 0)),
            pl.BlockSpec((NC, blk, d), lambda i: (0, i, 0)),
            pl.BlockSpec((NW, blk), lambda i: (0, i)),
            pl.BlockSpec((d, d), lambda i: (0, 0)),
            pl.BlockSpec((d, d), lambda i: (0, 0)),
            pl.BlockSpec((1, d), lambda i: (0, 0)),
        ],
        out_specs=pl.BlockSpec((blk, d), lambda i: (i, 0)),
        out_shape=jax.ShapeDtypeStruct((n, d), jnp.float32),
    )(h1, q, degp, w_self, w_neigh, b)


def kernel(x, edge_index, W_self1, W_neigh1, b1, W_self2, W_neigh2, b2):
    n, din = x.shape
    e = edge_index.shape[1]
    src = edge_index[0].astype(jnp.int32)
    dst = edge_index[1].astype(jnp.int32)

    blk = 2048
    n_pad = ((n + NS * 128 - 1) // (NS * 128)) * (NS * 128)
    xp = jnp.zeros((n_pad, din), jnp.float32).at[:n].set(x)

    p, degf = _make_sc_aggregate(n_pad, e, din, True)(xp, src, dst)
    degp = degf.reshape(NW, n_pad)
    h1 = _tc_layer1(xp, p, degp, W_self1, W_neigh1, b1.reshape(1, -1), blk)
    (q,) = _make_sc_aggregate(n_pad, e, din, False)(h1, src, dst)
    out = _tc_layer2(h1, q, degp, W_self2, W_neigh2, b2.reshape(1, -1), blk)
    return out[:n]


# speedup vs baseline: 1.0265x; 1.0244x over previous
import functools

import jax
import jax.numpy as jnp
from jax import lax
from jax.experimental import pallas as pl
from jax.experimental.pallas import tpu as pltpu
from jax.experimental.pallas import tpu_sc as plsc

NC = 2
NS = 16
NW = NC * NS
EDGE_CHUNK = 80


def _make_sc_aggregate(n_pad, n_edges, d, with_deg):
    rows_per_tile = n_pad // NS
    assert rows_per_tile % EDGE_CHUNK == 0
    edges_per_worker = n_edges // NW
    assert edges_per_worker % EDGE_CHUNK == 0
    n_chunks = edges_per_worker // EDGE_CHUNK
    assert n_chunks % 2 == 1 and n_chunks >= 3

    mesh = plsc.VectorSubcoreMesh(core_axis_name="c", subcore_axis_name="s")

    stage_dst = not with_deg

    out_type = [jax.ShapeDtypeStruct((NC, n_pad, d), jnp.float32)]
    scratch = [
        pltpu.VMEM((edges_per_worker,), jnp.int32),
        pltpu.VMEM((EDGE_CHUNK,), jnp.int32),
        pltpu.VMEM((EDGE_CHUNK,), jnp.int32),
        pltpu.VMEM((EDGE_CHUNK, d), jnp.float32),
        pltpu.VMEM((EDGE_CHUNK, d), jnp.float32),
        pltpu.VMEM_SHARED((n_pad, d), jnp.float32),
        pltpu.SemaphoreType.DMA,
        pltpu.SemaphoreType.DMA,
    ]
    if with_deg:
        out_type.append(jax.ShapeDtypeStruct((NW * n_pad,), jnp.float32))
        scratch.append(pltpu.VMEM((n_pad,), jnp.float32))
    if stage_dst:
        scratch.append(pltpu.VMEM((edges_per_worker,), jnp.int32))

    @functools.partial(
        pl.kernel, mesh=mesh, out_type=out_type, scratch_types=scratch,
        compiler_params=pltpu.CompilerParams(needs_layout_passes=False))
    def sc_aggregate(rows_hbm, src_hbm, dst_hbm, out_hbm, *rest):
        if with_deg:
            (deg_hbm, src_all, dst_a, dst_b, rows_a, rows_b,
             agg_sh, sem_a, sem_b, deg_v) = rest
            dst_all = None
        else:
            (src_all, dst_a, dst_b, rows_a, rows_b, agg_sh,
             sem_a, sem_b, dst_all) = rest
        c = lax.axis_index("c")
        s = lax.axis_index("s")
        w = c * NS + s
        base = w * edges_per_worker

        pltpu.sync_copy(src_hbm.at[pl.ds(base, edges_per_worker)], src_all)
        if stage_dst:
            pltpu.sync_copy(dst_hbm.at[pl.ds(base, edges_per_worker)],
                            dst_all)

        def _zrow(i, _):
            def _zcol(j, _):
                rows_a[i, pl.ds(j * 16, 16)] = jnp.zeros((16,), jnp.float32)
                return 0
            return lax.fori_loop(0, d // 16, _zcol, 0)
        lax.fori_loop(0, EDGE_CHUNK, _zrow, 0)

        def _zcopy(k, _):
            pltpu.sync_copy(
                rows_a,
                agg_sh.at[pl.ds(s * rows_per_tile + k * EDGE_CHUNK,
                                EDGE_CHUNK)])
            return 0
        lax.fori_loop(0, rows_per_tile // EDGE_CHUNK, _zcopy, 0)

        if with_deg:
            def _zdeg(i, _):
                deg_v[pl.ds(i * 16, 16)] = jnp.zeros((16,), jnp.float32)
                return 0
            lax.fori_loop(0, n_pad // 16, _zdeg, 0)
            ones16 = jnp.ones((16,), jnp.float32)

        plsc.subcore_barrier()

        def _gather(j, buf, sem):
            pltpu.async_copy(
                rows_hbm.at[src_all.at[pl.ds(j * EDGE_CHUNK, EDGE_CHUNK)]],
                buf, sem)

        def _wait(buf, sem):
            pltpu.make_async_copy(rows_hbm.at[pl.ds(0, EDGE_CHUNK)], buf,
                                  sem).wait()

        def _scatter(dst_v, buf):
            pltpu.sync_copy(buf, agg_sh.at[dst_v], add=True)

        def _load_dst(j, dst_v):
            if stage_dst:
                for k in range(EDGE_CHUNK // 16):
                    dst_v[pl.ds(k * 16, 16)] = (
                        dst_all[pl.ds(j * EDGE_CHUNK + k * 16, 16)])
            else:
                pltpu.sync_copy(dst_hbm.at[pl.ds(base + j * EDGE_CHUNK,
                                                 EDGE_CHUNK)], dst_v)
            if with_deg:
                for k in range(EDGE_CHUNK // 16):
                    idx16 = dst_v[pl.ds(k * 16, 16)]
                    plsc.addupdate_scatter(deg_v, [idx16], ones16)

        _gather(0, rows_a, sem_a)
        _load_dst(0, dst_a)

        def _pair(gg, _):
            j0 = 2 * gg
            _gather(j0 + 1, rows_b, sem_b)
            _load_dst(j0 + 1, dst_b)
            _wait(rows_a, sem_a)
            _scatter(dst_a, rows_a)
            _gather(j0 + 2, rows_a, sem_a)
            _load_dst(j0 + 2, dst_a)
            _wait(rows_b, sem_b)
            _scatter(dst_b, rows_b)
            return 0
        lax.fori_loop(0, (n_chunks - 1) // 2, _pair, 0)

        _wait(rows_a, sem_a)
        _scatter(dst_a, rows_a)

        if with_deg:
            pltpu.sync_copy(deg_v, deg_hbm.at[pl.ds(w * n_pad, n_pad)])

        plsc.subcore_barrier()

        pltpu.sync_copy(agg_sh.at[pl.ds(s * rows_per_tile, rows_per_tile)],
                        out_hbm.at[c, pl.ds(s * rows_per_tile, rows_per_tile)])

    return sc_aggregate


def _deg_column(dp):
    dpt = jnp.transpose(dp)
    return jnp.maximum(jnp.sum(dpt, axis=1, keepdims=True), 1.0)


def _tc_layer1(x, p, degp, w_self, w_neigh, b, blk):
    n, din = x.shape

    def body(x_ref, p_ref, dp_ref, ws_ref, wn_ref, b_ref, h_ref):
        agg = p_ref[0] + p_ref[1]
        hn = agg / _deg_column(dp_ref[...])
        h = (jnp.dot(x_ref[...], ws_ref[...], preferred_element_type=jnp.float32)
             + jnp.dot(hn, wn_ref[...], preferred_element_type=jnp.float32)
             + b_ref[...])
        h_ref[...] = jnp.maximum(h, 0.0)

    return pl.pallas_call(
        body,
        grid=(pl.cdiv(n, blk),),
        in_specs=[
            pl.BlockSpec((blk, din), lambda i: (i, 0)),
            pl.BlockSpec((NC, blk, din), lambda i: (0, i, 0)),
            pl.BlockSpec((NW, blk), lambda i: (0, i)),
            pl.BlockSpec((din, din), lambda i: (0, 0)),
            pl.BlockSpec((din, din), lambda i: (0, 0)),
            pl.BlockSpec((1, din), lambda i: (0, 0)),
        ],
        out_specs=pl.BlockSpec((blk, din), lambda i: (i, 0)),
        out_shape=jax.ShapeDtypeStruct((n, din), jnp.float32),
    )(x, p, degp, w_self, w_neigh, b)


def _tc_layer2(h1, q, degp, w_self, w_neigh, b, blk):
    n, d = h1.shape

    def body(h_ref, q_ref, dp_ref, ws_ref, wn_ref, b_ref, o_ref):
        hn = (q_ref[0] + q_ref[1]) / _deg_column(dp_ref[...])
        o_ref[...] = (
            jnp.dot(h_ref[...], ws_ref[...], preferred_element_type=jnp.float32)
            + jnp.dot(hn, wn_ref[...], preferred_element_type=jnp.float32)
            + b_ref[...])

    return pl.pallas_call(
        body,
        grid=(pl.cdiv(n, blk),),
        in_specs=[
            pl.BlockSpec((blk, d), lambda i: (i, 0)),
            pl.BlockSpec((NC, blk, d), lambda i: (0, i, 0)),
            pl.BlockSpec((NW, blk), lambda i: (0, i)),
            pl.BlockSpec((d, d), lambda i: (0, 0)),
            pl.BlockSpec((d, d), lambda i: (0, 0)),
            pl.BlockSpec((1, d), lambda i: (0, 0)),
        ],
        out_specs=pl.BlockSpec((blk, d), lambda i: (i, 0)),
        out_shape=jax.ShapeDtypeStruct((n, d), jnp.float32),
    )(h1, q, degp, w_self, w_neigh, b)


def kernel(x, edge_index, W_self1, W_neigh1, b1, W_self2, W_neigh2, b2):
    n, din = x.shape
    e = edge_index.shape[1]
    src = edge_index[0].astype(jnp.int32)
    dst = edge_index[1].astype(jnp.int32)

    blk = 2048
    n_pad = ((n + NS * 128 - 1) // (NS * 128)) * (NS * 128)

    p, degf = _make_sc_aggregate(n_pad, e, din, True)(x, src, dst)
    degp = degf.reshape(NW, n_pad)
    h1 = _tc_layer1(x, p, degp, W_self1, W_neigh1, b1.reshape(1, -1), blk)
    (q,) = _make_sc_aggregate(n_pad, e, din, False)(h1, src, dst)
    out = _tc_layer2(h1, q, degp, W_self2, W_neigh2, b2.reshape(1, -1), blk)
    return out


# speedup vs baseline: 1.0727x; 1.0451x over previous
import functools

import jax
import jax.numpy as jnp
from jax import lax
from jax.experimental import pallas as pl
from jax.experimental.pallas import tpu as pltpu
from jax.experimental.pallas import tpu_sc as plsc

NC = 2
NS = 16
NW = NC * NS
EDGE_CHUNK = 80


def _make_sc_aggregate(n_pad, n_edges, d, with_deg):
    rows_per_tile = n_pad // NS
    assert rows_per_tile % EDGE_CHUNK == 0
    edges_per_worker = n_edges // NW
    assert edges_per_worker % EDGE_CHUNK == 0
    n_chunks = edges_per_worker // EDGE_CHUNK
    assert n_chunks % 2 == 1 and n_chunks >= 3

    mesh = plsc.VectorSubcoreMesh(core_axis_name="c", subcore_axis_name="s")

    stage_dst = not with_deg

    out_type = [jax.ShapeDtypeStruct((NC, n_pad, d), jnp.float32)]
    scratch = [
        pltpu.VMEM((edges_per_worker,), jnp.int32),
        pltpu.VMEM((EDGE_CHUNK,), jnp.int32),
        pltpu.VMEM((EDGE_CHUNK,), jnp.int32),
        pltpu.VMEM((EDGE_CHUNK, d), jnp.float32),
        pltpu.VMEM((EDGE_CHUNK, d), jnp.float32),
        pltpu.VMEM_SHARED((n_pad, d), jnp.float32),
        pltpu.SemaphoreType.DMA,
        pltpu.SemaphoreType.DMA,
    ]
    if with_deg:
        out_type.append(jax.ShapeDtypeStruct((NW * n_pad,), jnp.float32))
        scratch.append(pltpu.VMEM((n_pad,), jnp.float32))
        scratch.append(pltpu.SemaphoreType.DMA)
        scratch.append(pltpu.SemaphoreType.DMA)
    if stage_dst:
        scratch.append(pltpu.VMEM((edges_per_worker,), jnp.int32))

    @functools.partial(
        pl.kernel, mesh=mesh, out_type=out_type, scratch_types=scratch,
        compiler_params=pltpu.CompilerParams(needs_layout_passes=False))
    def sc_aggregate(rows_hbm, src_hbm, dst_hbm, out_hbm, *rest):
        if with_deg:
            (deg_hbm, src_all, dst_a, dst_b, rows_a, rows_b,
             agg_sh, sem_a, sem_b, deg_v, dsem_a, dsem_b) = rest
            dst_all = None
        else:
            (src_all, dst_a, dst_b, rows_a, rows_b, agg_sh,
             sem_a, sem_b, dst_all) = rest
            dsem_a = dsem_b = None
        c = lax.axis_index("c")
        s = lax.axis_index("s")
        w = c * NS + s
        base = w * edges_per_worker

        pltpu.sync_copy(src_hbm.at[pl.ds(base, edges_per_worker)], src_all)
        if stage_dst:
            pltpu.sync_copy(dst_hbm.at[pl.ds(base, edges_per_worker)],
                            dst_all)

        def _zrow(i, _):
            def _zcol(j, _):
                rows_a[i, pl.ds(j * 16, 16)] = jnp.zeros((16,), jnp.float32)
                return 0
            return lax.fori_loop(0, d // 16, _zcol, 0)
        lax.fori_loop(0, EDGE_CHUNK, _zrow, 0)

        def _zcopy(k, _):
            pltpu.sync_copy(
                rows_a,
                agg_sh.at[pl.ds(s * rows_per_tile + k * EDGE_CHUNK,
                                EDGE_CHUNK)])
            return 0
        lax.fori_loop(0, rows_per_tile // EDGE_CHUNK, _zcopy, 0)

        if with_deg:
            def _zdeg(i, _):
                deg_v[pl.ds(i * 16, 16)] = jnp.zeros((16,), jnp.float32)
                return 0
            lax.fori_loop(0, n_pad // 16, _zdeg, 0)
            ones16 = jnp.ones((16,), jnp.float32)

        plsc.subcore_barrier()

        def _gather(j, buf, sem):
            pltpu.async_copy(
                rows_hbm.at[src_all.at[pl.ds(j * EDGE_CHUNK, EDGE_CHUNK)]],
                buf, sem)

        def _wait(buf, sem):
            pltpu.make_async_copy(rows_hbm.at[pl.ds(0, EDGE_CHUNK)], buf,
                                  sem).wait()

        def _scatter(dst_v, buf):
            pltpu.sync_copy(buf, agg_sh.at[dst_v], add=True)

        def _issue_dst(j, dst_v, dsem):
            if stage_dst:
                for k in range(EDGE_CHUNK // 16):
                    dst_v[pl.ds(k * 16, 16)] = (
                        dst_all[pl.ds(j * EDGE_CHUNK + k * 16, 16)])
            else:
                pltpu.async_copy(
                    dst_hbm.at[pl.ds(base + j * EDGE_CHUNK, EDGE_CHUNK)],
                    dst_v, dsem)

        def _finish_dst(dst_v, dsem):
            if not stage_dst:
                pltpu.make_async_copy(dst_hbm.at[pl.ds(0, EDGE_CHUNK)],
                                      dst_v, dsem).wait()
            if with_deg:
                for k in range(EDGE_CHUNK // 16):
                    idx16 = dst_v[pl.ds(k * 16, 16)]
                    plsc.addupdate_scatter(deg_v, [idx16], ones16)

        _gather(0, rows_a, sem_a)
        _issue_dst(0, dst_a, dsem_a)

        def _pair(gg, _):
            j0 = 2 * gg
            _gather(j0 + 1, rows_b, sem_b)
            _issue_dst(j0 + 1, dst_b, dsem_b)
            _wait(rows_a, sem_a)
            _finish_dst(dst_a, dsem_a)
            _scatter(dst_a, rows_a)
            _gather(j0 + 2, rows_a, sem_a)
            _issue_dst(j0 + 2, dst_a, dsem_a)
            _wait(rows_b, sem_b)
            _finish_dst(dst_b, dsem_b)
            _scatter(dst_b, rows_b)
            return 0
        lax.fori_loop(0, (n_chunks - 1) // 2, _pair, 0)

        _wait(rows_a, sem_a)
        _finish_dst(dst_a, dsem_a)
        _scatter(dst_a, rows_a)

        if with_deg:
            pltpu.sync_copy(deg_v, deg_hbm.at[pl.ds(w * n_pad, n_pad)])

        plsc.subcore_barrier()

        pltpu.sync_copy(agg_sh.at[pl.ds(s * rows_per_tile, rows_per_tile)],
                        out_hbm.at[c, pl.ds(s * rows_per_tile, rows_per_tile)])

    return sc_aggregate


def _deg_column(dp):
    dpt = jnp.transpose(dp)
    return jnp.maximum(jnp.sum(dpt, axis=1, keepdims=True), 1.0)


def _tc_layer1(x, p, degp, w_self, w_neigh, b, blk):
    n, din = x.shape

    def body(x_ref, p_ref, dp_ref, ws_ref, wn_ref, b_ref, h_ref):
        agg = p_ref[0] + p_ref[1]
        hn = agg / _deg_column(dp_ref[...])
        h = (jnp.dot(x_ref[...], ws_ref[...], preferred_element_type=jnp.float32)
             + jnp.dot(hn, wn_ref[...], preferred_element_type=jnp.float32)
             + b_ref[...])
        h_ref[...] = jnp.maximum(h, 0.0)

    return pl.pallas_call(
        body,
        grid=(pl.cdiv(n, blk),),
        in_specs=[
            pl.BlockSpec((blk, din), lambda i: (i, 0)),
            pl.BlockSpec((NC, blk, din), lambda i: (0, i, 0)),
            pl.BlockSpec((NW, blk), lambda i: (0, i)),
            pl.BlockSpec((din, din), lambda i: (0, 0)),
            pl.BlockSpec((din, din), lambda i: (0, 0)),
            pl.BlockSpec((1, din), lambda i: (0, 0)),
        ],
        out_specs=pl.BlockSpec((blk, din), lambda i: (i, 0)),
        out_shape=jax.ShapeDtypeStruct((n, din), jnp.float32),
    )(x, p, degp, w_self, w_neigh, b)


def _tc_layer2(h1, q, degp, w_self, w_neigh, b, blk):
    n, d = h1.shape

    def body(h_ref, q_ref, dp_ref, ws_ref, wn_ref, b_ref, o_ref):
        hn = (q_ref[0] + q_ref[1]) / _deg_column(dp_ref[...])
        o_ref[...] = (
            jnp.dot(h_ref[...], ws_ref[...], preferred_element_type=jnp.float32)
            + jnp.dot(hn, wn_ref[...], preferred_element_type=jnp.float32)
            + b_ref[...])

    return pl.pallas_call(
        body,
        grid=(pl.cdiv(n, blk),),
        in_specs=[
            pl.BlockSpec((blk, d), lambda i: (i, 0)),
            pl.BlockSpec((NC, blk, d), lambda i: (0, i, 0)),
            pl.BlockSpec((NW, blk), lambda i: (0, i)),
            pl.BlockSpec((d, d), lambda i: (0, 0)),
            pl.BlockSpec((d, d), lambda i: (0, 0)),
            pl.BlockSpec((1, d), lambda i: (0, 0)),
        ],
        out_specs=pl.BlockSpec((blk, d), lambda i: (i, 0)),
        out_shape=jax.ShapeDtypeStruct((n, d), jnp.float32),
    )(h1, q, degp, w_self, w_neigh, b)


def kernel(x, edge_index, W_self1, W_neigh1, b1, W_self2, W_neigh2, b2):
    n, din = x.shape
    e = edge_index.shape[1]
    src = edge_index[0].astype(jnp.int32)
    dst = edge_index[1].astype(jnp.int32)

    blk = 2048
    n_pad = ((n + NS * 128 - 1) // (NS * 128)) * (NS * 128)

    p, degf = _make_sc_aggregate(n_pad, e, din, True)(x, src, dst)
    degp = degf.reshape(NW, n_pad)
    h1 = _tc_layer1(x, p, degp, W_self1, W_neigh1, b1.reshape(1, -1), blk)
    (q,) = _make_sc_aggregate(n_pad, e, din, False)(h1, src, dst)
    out = _tc_layer2(h1, q, degp, W_self2, W_neigh2, b2.reshape(1, -1), blk)
    return out
